# Initial kernel scaffold; baseline (speedup 1.0000x reference)
import jax
import jax.numpy as jnp
from jax.experimental import pallas as pl


def kernel(cat_fea_list, dense_features, table, W1_ctr, b1_ctr, W2_ctr, b2_ctr, W1_cvr, b1_cvr, W2_cvr, b2_cvr):
    raise NotImplementedError("write your pallas kernel here")


# speedup vs baseline: 2.6148x; 2.6148x over previous
import functools

import jax
import jax.numpy as jnp
from jax import lax
from jax.experimental import pallas as pl
from jax.experimental.pallas import tpu as pltpu
from jax.experimental.pallas import tpu_sc as plsc

B, F, V, D = 4096, 26, 100000, 128
DENSE, H = 13, 1024
KE = F * D
N = B * F
DPAD = 16

_NC, _NS = 2, 16
NW = _NC * _NS
PER_W = N // NW
CHUNK = 128
N_CH = PER_W // CHUNK

BM = 512


def _gather_body(idx_hbm, table_hbm, out_hbm, idx_v, rows_v, sem):
    wid = lax.axis_index("s") * _NC + lax.axis_index("c")
    base = wid * PER_W

    def body(c, carry):
        b0 = pl.multiple_of(base + c * CHUNK, CHUNK)
        pltpu.sync_copy(idx_hbm.at[pl.ds(b0, CHUNK)], idx_v)
        pltpu.async_copy(table_hbm.at[idx_v], rows_v, sem).wait()
        pltpu.sync_copy(rows_v, out_hbm.at[pl.ds(b0, CHUNK)])
        return carry

    lax.fori_loop(0, N_CH, body, 0)


def _sc_gather(idx_flat, table):
    mesh = plsc.VectorSubcoreMesh(core_axis_name="c", subcore_axis_name="s")
    f = pl.kernel(
        _gather_body,
        out_type=jax.ShapeDtypeStruct((N, D), jnp.float32),
        mesh=mesh,
        scratch_types=[
            pltpu.VMEM((CHUNK,), jnp.int32),
            pltpu.VMEM((CHUNK, D), jnp.float32),
            pltpu.SemaphoreType.DMA,
        ],
    )
    return f(idx_flat, table)


def _towers_body(fc_ref, dn_ref, w1c_ref, w1v_ref, wdc_ref, wdv_ref,
                 b1c_ref, b1v_ref, w2c_ref, w2v_ref, b2_ref,
                 octr_ref, ocvr_ref):
    x = fc_ref[...].astype(jnp.bfloat16)
    xd = dn_ref[...].astype(jnp.bfloat16)
    for w1, wd, b1, w2, j, oref in (
        (w1c_ref, wdc_ref, b1c_ref, w2c_ref, 0, octr_ref),
        (w1v_ref, wdv_ref, b1v_ref, w2v_ref, 1, ocvr_ref),
    ):
        acc = jnp.dot(x, w1[...], preferred_element_type=jnp.float32)
        acc = acc + jnp.dot(xd, wd[...], preferred_element_type=jnp.float32)
        h = jnp.maximum(acc + b1[...], 0.0)
        logit = jnp.sum(h * w2[...], axis=1, keepdims=True) + b2_ref[0, j]
        oref[...] = 1.0 / (1.0 + jnp.exp(-logit))


def _towers(fc, densep, w1c, w1v, wdc, wdv, b1c, b1v, w2c, w2v, b2):
    nb = B // BM
    rep = lambda b: (0, 0)
    return pl.pallas_call(
        _towers_body,
        grid=(nb,),
        in_specs=[
            pl.BlockSpec((BM, KE), lambda b: (b, 0)),
            pl.BlockSpec((BM, DPAD), lambda b: (b, 0)),
            pl.BlockSpec((KE, H), rep),
            pl.BlockSpec((KE, H), rep),
            pl.BlockSpec((DPAD, H), rep),
            pl.BlockSpec((DPAD, H), rep),
            pl.BlockSpec((1, H), rep),
            pl.BlockSpec((1, H), rep),
            pl.BlockSpec((1, H), rep),
            pl.BlockSpec((1, H), rep),
            pl.BlockSpec(memory_space=pltpu.SMEM),
        ],
        out_specs=[
            pl.BlockSpec((BM, 1), lambda b: (b, 0)),
            pl.BlockSpec((BM, 1), lambda b: (b, 0)),
        ],
        out_shape=[
            jax.ShapeDtypeStruct((B, 1), jnp.float32),
            jax.ShapeDtypeStruct((B, 1), jnp.float32),
        ],
    )(fc, densep, w1c, w1v, wdc, wdv, b1c, b1v, w2c, w2v, b2)


def kernel(cat_fea_list, dense_features, table,
           W1_ctr, b1_ctr, W2_ctr, b2_ctr,
           W1_cvr, b1_cvr, W2_cvr, b2_cvr):
    idx_flat = cat_fea_list.reshape(-1)
    emb = _sc_gather(idx_flat, table)
    fc = emb.reshape(B, KE)
    densep = jnp.pad(dense_features, ((0, 0), (0, DPAD - DENSE)))
    w1c = W1_ctr[:KE].astype(jnp.bfloat16)
    w1v = W1_cvr[:KE].astype(jnp.bfloat16)
    wdc = jnp.pad(W1_ctr[KE:], ((0, DPAD - DENSE), (0, 0))).astype(jnp.bfloat16)
    wdv = jnp.pad(W1_cvr[KE:], ((0, DPAD - DENSE), (0, 0))).astype(jnp.bfloat16)
    b1c = b1_ctr.reshape(1, H)
    b1v = b1_cvr.reshape(1, H)
    w2c = W2_ctr.reshape(1, H)
    w2v = W2_cvr.reshape(1, H)
    b2 = jnp.concatenate([b2_ctr, b2_cvr]).reshape(1, 2)
    octr, ocvr = _towers(fc, densep, w1c, w1v, wdc, wdv,
                         b1c, b1v, w2c, w2v, b2)
    return octr[:, 0], ocvr[:, 0]


# speedup vs baseline: 2.7914x; 1.0675x over previous
import functools

import jax
import jax.numpy as jnp
from jax import lax
from jax.experimental import pallas as pl
from jax.experimental.pallas import tpu as pltpu
from jax.experimental.pallas import tpu_sc as plsc

B, F, V, D = 4096, 26, 100000, 128
DENSE, H = 13, 1024
KE = F * D
N = B * F
DPAD = 16

_NC, _NS = 2, 16
NW = _NC * _NS
PER_W = N // NW
CHUNK = 128
N_CH = PER_W // CHUNK

BM = 512


def _gather_body(idx_hbm, table_hbm, out_hbm, idx_v, rows0, rows1, sem0, sem1):
    wid = lax.axis_index("s") * _NC + lax.axis_index("c")
    base = wid * PER_W
    pltpu.sync_copy(idx_hbm.at[wid], idx_v)
    pltpu.async_copy(table_hbm.at[idx_v.at[0]], rows0, sem0)
    pltpu.async_copy(table_hbm.at[idx_v.at[1]], rows1, sem1)

    def body(i, carry):
        for b, (buf, sem) in enumerate(((rows0, sem0), (rows1, sem1))):
            c = i * 2 + b
            pltpu.make_async_copy(table_hbm.at[idx_v.at[c]], buf, sem).wait()
            b0 = pl.multiple_of(base + c * CHUNK, CHUNK)
            pltpu.sync_copy(buf, out_hbm.at[pl.ds(b0, CHUNK)])

            @pl.when(c + 2 < N_CH)
            def _():
                pltpu.async_copy(
                    table_hbm.at[idx_v.at[jnp.minimum(c + 2, N_CH - 1)]],
                    buf, sem)

        return carry

    lax.fori_loop(0, N_CH // 2, body, 0)


def _sc_gather(idx2, table):
    mesh = plsc.VectorSubcoreMesh(core_axis_name="c", subcore_axis_name="s")
    f = pl.kernel(
        _gather_body,
        out_type=jax.ShapeDtypeStruct((N, D), jnp.float32),
        mesh=mesh,
        scratch_types=[
            pltpu.VMEM((N_CH, CHUNK), jnp.int32),
            pltpu.VMEM((CHUNK, D), jnp.float32),
            pltpu.VMEM((CHUNK, D), jnp.float32),
            pltpu.SemaphoreType.DMA,
            pltpu.SemaphoreType.DMA,
        ],
    )
    return f(idx2, table)


def _towers_body(fc_ref, dn_ref, w1c_ref, w1v_ref, wdc_ref, wdv_ref,
                 b1c_ref, b1v_ref, w2c_ref, w2v_ref, b2_ref,
                 octr_ref, ocvr_ref):
    x = fc_ref[...].astype(jnp.bfloat16)
    xd = dn_ref[...].astype(jnp.bfloat16)
    for w1, wd, b1, w2, j, oref in (
        (w1c_ref, wdc_ref, b1c_ref, w2c_ref, 0, octr_ref),
        (w1v_ref, wdv_ref, b1v_ref, w2v_ref, 1, ocvr_ref),
    ):
        acc = jnp.dot(x, w1[...], preferred_element_type=jnp.float32)
        acc = acc + jnp.dot(xd, wd[...], preferred_element_type=jnp.float32)
        h = jnp.maximum(acc + b1[...], 0.0)
        logit = jnp.sum(h * w2[...], axis=1, keepdims=True) + b2_ref[0, j]
        oref[...] = 1.0 / (1.0 + jnp.exp(-logit))


def _towers(fc, densep, w1c, w1v, wdc, wdv, b1c, b1v, w2c, w2v, b2):
    nb = B // BM
    rep = lambda b: (0, 0)
    return pl.pallas_call(
        _towers_body,
        grid=(nb,),
        in_specs=[
            pl.BlockSpec((BM, KE), lambda b: (b, 0)),
            pl.BlockSpec((BM, DPAD), lambda b: (b, 0)),
            pl.BlockSpec((KE, H), rep),
            pl.BlockSpec((KE, H), rep),
            pl.BlockSpec((DPAD, H), rep),
            pl.BlockSpec((DPAD, H), rep),
            pl.BlockSpec((1, H), rep),
            pl.BlockSpec((1, H), rep),
            pl.BlockSpec((1, H), rep),
            pl.BlockSpec((1, H), rep),
            pl.BlockSpec(memory_space=pltpu.SMEM),
        ],
        out_specs=[
            pl.BlockSpec((BM, 1), lambda b: (b, 0)),
            pl.BlockSpec((BM, 1), lambda b: (b, 0)),
        ],
        out_shape=[
            jax.ShapeDtypeStruct((B, 1), jnp.float32),
            jax.ShapeDtypeStruct((B, 1), jnp.float32),
        ],
    )(fc, densep, w1c, w1v, wdc, wdv, b1c, b1v, w2c, w2v, b2)


def kernel(cat_fea_list, dense_features, table,
           W1_ctr, b1_ctr, W2_ctr, b2_ctr,
           W1_cvr, b1_cvr, W2_cvr, b2_cvr):
    idx2 = cat_fea_list.reshape(NW, N_CH, CHUNK)
    emb = _sc_gather(idx2, table)
    fc = emb.reshape(B, KE)
    densep = jnp.pad(dense_features, ((0, 0), (0, DPAD - DENSE)))
    w1c = W1_ctr[:KE].astype(jnp.bfloat16)
    w1v = W1_cvr[:KE].astype(jnp.bfloat16)
    wdc = jnp.pad(W1_ctr[KE:], ((0, DPAD - DENSE), (0, 0))).astype(jnp.bfloat16)
    wdv = jnp.pad(W1_cvr[KE:], ((0, DPAD - DENSE), (0, 0))).astype(jnp.bfloat16)
    b1c = b1_ctr.reshape(1, H)
    b1v = b1_cvr.reshape(1, H)
    w2c = W2_ctr.reshape(1, H)
    w2v = W2_cvr.reshape(1, H)
    b2 = jnp.concatenate([b2_ctr, b2_cvr]).reshape(1, 2)
    octr, ocvr = _towers(fc, densep, w1c, w1v, wdc, wdv,
                         b1c, b1v, w2c, w2v, b2)
    return octr[:, 0], ocvr[:, 0]


# speedup vs baseline: 3.1143x; 1.1157x over previous
import jax
import jax.numpy as jnp
from jax import lax
from jax.experimental import pallas as pl
from jax.experimental.pallas import tpu as pltpu
from jax.experimental.pallas import tpu_sc as plsc

B, F, V, D = 4096, 26, 100000, 128
DENSE, H = 13, 1024
KE = F * D
N = B * F

_NC, _NS = 2, 16
NW = _NC * _NS
PER_W = N // NW
CHUNK = 104
N_CH = PER_W // CHUNK
NBUF = 4

BM = 512


def _gather_body(idx_hbm, table_hbm, out_hbm, idx_v,
                 b0_v, b1_v, b2_v, b3_v,
                 g0, g1, g2, g3, w0, w1, w2, w3):
    wid = lax.axis_index("s") * _NC + lax.axis_index("c")
    base = wid * PER_W
    bufs = (b0_v, b1_v, b2_v, b3_v)
    gsems = (g0, g1, g2, g3)
    wsems = (w0, w1, w2, w3)

    def out_at(c):
        return out_hbm.at[pl.ds(pl.multiple_of(base + c * CHUNK, 8), CHUNK)]

    pltpu.sync_copy(idx_hbm.at[wid], idx_v)
    pltpu.async_copy(table_hbm.at[idx_v.at[0]], bufs[0], gsems[0])
    pltpu.async_copy(table_hbm.at[idx_v.at[1]], bufs[1], gsems[1])

    def body(i, carry):
        for b in range(NBUF):
            c = i * NBUF + b
            sp = (b + 2) % NBUF

            @pl.when(c >= 2)
            def _():
                pltpu.make_async_copy(bufs[sp], out_at(c - 2), wsems[sp]).wait()

            @pl.when(c + 2 < N_CH)
            def _():
                pltpu.async_copy(
                    table_hbm.at[idx_v.at[jnp.minimum(c + 2, N_CH - 1)]],
                    bufs[sp], gsems[sp])

            pltpu.make_async_copy(table_hbm.at[idx_v.at[c]],
                                  bufs[b], gsems[b]).wait()
            pltpu.async_copy(bufs[b], out_at(c), wsems[b])
        return carry

    lax.fori_loop(0, N_CH // NBUF, body, 0)
    pltpu.make_async_copy(bufs[(N_CH - 2) % NBUF], out_at(N_CH - 2),
                          wsems[(N_CH - 2) % NBUF]).wait()
    pltpu.make_async_copy(bufs[(N_CH - 1) % NBUF], out_at(N_CH - 1),
                          wsems[(N_CH - 1) % NBUF]).wait()


def _sc_gather(idx3, table):
    mesh = plsc.VectorSubcoreMesh(core_axis_name="c", subcore_axis_name="s")
    f = pl.kernel(
        _gather_body,
        out_type=jax.ShapeDtypeStruct((N, D), jnp.float32),
        mesh=mesh,
        scratch_types=(
            [pltpu.VMEM((N_CH, CHUNK), jnp.int32)]
            + [pltpu.VMEM((CHUNK, D), jnp.float32)] * NBUF
            + [pltpu.SemaphoreType.DMA] * (2 * NBUF)
        ),
    )
    return f(idx3, table)


def _towers_body(fc_ref, dn_ref, w1c_ref, w1v_ref,
                 b1c_ref, b1v_ref, w2c_ref, w2v_ref, b2_ref,
                 octr_ref, ocvr_ref):
    accs = []
    for w1 in (w1c_ref, w1v_ref):
        acc = None
        for f in range(F):
            xf = fc_ref[f].astype(jnp.bfloat16)
            wf = w1[pl.ds(f * D, D), :].astype(jnp.bfloat16)
            d = jnp.dot(xf, wf, preferred_element_type=jnp.float32)
            acc = d if acc is None else acc + d
        xd = dn_ref[...].astype(jnp.bfloat16)
        wd = w1[pl.ds(KE, DENSE), :].astype(jnp.bfloat16)
        acc = acc + jnp.dot(xd, wd, preferred_element_type=jnp.float32)
        accs.append(acc)
    for j, (acc, b1, w2, oref) in enumerate((
        (accs[0], b1c_ref, w2c_ref, octr_ref),
        (accs[1], b1v_ref, w2v_ref, ocvr_ref),
    )):
        h = jnp.maximum(acc + b1[...], 0.0)
        logit = jnp.sum(h * w2[...], axis=1, keepdims=True) + b2_ref[0, j]
        oref[...] = 1.0 / (1.0 + jnp.exp(-logit))


def _towers(fc3, dense, w1c, w1v, b1c, b1v, w2c, w2v, b2):
    nb = B // BM
    rep = lambda b: (0, 0)
    return pl.pallas_call(
        _towers_body,
        grid=(nb,),
        in_specs=[
            pl.BlockSpec((F, BM, D), lambda b: (0, b, 0)),
            pl.BlockSpec((BM, DENSE), lambda b: (b, 0)),
            pl.BlockSpec((KE + DENSE, H), rep),
            pl.BlockSpec((KE + DENSE, H), rep),
            pl.BlockSpec((1, H), rep),
            pl.BlockSpec((1, H), rep),
            pl.BlockSpec((1, H), rep),
            pl.BlockSpec((1, H), rep),
            pl.BlockSpec(memory_space=pltpu.SMEM),
        ],
        out_specs=[
            pl.BlockSpec((BM, 1), lambda b: (b, 0)),
            pl.BlockSpec((BM, 1), lambda b: (b, 0)),
        ],
        out_shape=[
            jax.ShapeDtypeStruct((B, 1), jnp.float32),
            jax.ShapeDtypeStruct((B, 1), jnp.float32),
        ],
    )(fc3, dense, w1c, w1v, b1c, b1v, w2c, w2v, b2)


def kernel(cat_fea_list, dense_features, table,
           W1_ctr, b1_ctr, W2_ctr, b2_ctr,
           W1_cvr, b1_cvr, W2_cvr, b2_cvr):
    idx3 = cat_fea_list.T.reshape(NW, N_CH, CHUNK)
    emb = _sc_gather(idx3, table)
    fc3 = emb.reshape(F, B, D)
    b2 = jnp.concatenate([b2_ctr, b2_cvr]).reshape(1, 2)
    octr, ocvr = _towers(fc3, dense_features, W1_ctr, W1_cvr,
                         b1_ctr.reshape(1, H), b1_cvr.reshape(1, H),
                         W2_ctr.reshape(1, H), W2_cvr.reshape(1, H), b2)
    return octr[:, 0], ocvr[:, 0]
